# Initial kernel scaffold; baseline (speedup 1.0000x reference)
import functools

import jax
import jax.numpy as jnp
from jax import lax
from jax.experimental import pallas as pl
from jax.experimental.pallas import tpu as pltpu
from jax.experimental.pallas import tpu_sc as plsc

N_CLASSES = 1000
N_EMB = 64
BATCH = 16384
ACC_ROWS = 1024
CNT_W = 16
NW = 32
ROWS_PER_TILE = BATCH // NW
CHUNK = 128
N_CHUNKS = ROWS_PER_TILE // CHUNK
ZROWS = ACC_ROWS // 16

_mesh = plsc.VectorSubcoreMesh(core_axis_name="c", subcore_axis_name="s")


@functools.partial(
    pl.kernel,
    mesh=_mesh,
    out_type=[
        jax.ShapeDtypeStruct((2, ACC_ROWS, N_EMB), jnp.float32),
        jax.ShapeDtypeStruct((2, ACC_ROWS, CNT_W), jnp.float32),
    ],
    scratch_types=[
        pltpu.VMEM((ROWS_PER_TILE, N_EMB), jnp.float32),
        pltpu.VMEM((N_CHUNKS, CHUNK), jnp.int32),
        pltpu.VMEM((CHUNK, CNT_W), jnp.float32),
        pltpu.VMEM((ZROWS, N_EMB), jnp.float32),
        pltpu.VMEM((ZROWS, CNT_W), jnp.float32),
        pltpu.VMEM_SHARED((ACC_ROWS, N_EMB), jnp.float32),
        pltpu.VMEM_SHARED((ACC_ROWS, CNT_W), jnp.float32),
    ],
)
def _scatter_add(x_hbm, y_hbm, sums_hbm, cnts_hbm,
                 x_v, y_v, ones_v, zs_v, zc_v, acc_s, acc_c):
    cid = lax.axis_index("c")
    sid = lax.axis_index("s")
    wid = cid * 16 + sid

    zero16 = jnp.zeros((16,), jnp.float32)
    one16 = jnp.ones((16,), jnp.float32)

    def _init_rows(i, _):
        for k in range(N_EMB // 16):
            zs_v[i, pl.ds(k * 16, 16)] = zero16
        zc_v[i, :] = zero16
        return 0

    lax.fori_loop(0, ZROWS, _init_rows, 0)

    def _init_ones(i, _):
        ones_v[i, :] = one16
        return 0

    lax.fori_loop(0, CHUNK, _init_ones, 0)

    pltpu.sync_copy(zs_v, acc_s.at[pl.ds(sid * ZROWS, ZROWS)])
    pltpu.sync_copy(zc_v, acc_c.at[pl.ds(sid * ZROWS, ZROWS)])
    plsc.subcore_barrier()

    pltpu.sync_copy(y_hbm.at[pl.ds(wid * N_CHUNKS, N_CHUNKS)], y_v)
    pltpu.sync_copy(x_hbm.at[pl.ds(wid * ROWS_PER_TILE, ROWS_PER_TILE)], x_v)

    for j in range(N_CHUNKS):
        pltpu.sync_copy(x_v.at[pl.ds(j * CHUNK, CHUNK)],
                        acc_s.at[y_v.at[j]], add=True)
        pltpu.sync_copy(ones_v, acc_c.at[y_v.at[j]], add=True)
    plsc.subcore_barrier()

    pltpu.sync_copy(acc_s.at[pl.ds(sid * ZROWS, ZROWS)],
                    sums_hbm.at[cid, pl.ds(sid * ZROWS, ZROWS)])
    pltpu.sync_copy(acc_c.at[pl.ds(sid * ZROWS, ZROWS)],
                    cnts_hbm.at[cid, pl.ds(sid * ZROWS, ZROWS)])


def _finalize_body(nbt_ref, sums_ref, cnts_ref, centers_ref, out_ref):
    s = sums_ref[0] + sums_ref[1]
    c = cnts_ref[0] + cnts_ref[1]
    s = s[:N_CLASSES]
    cnt = c[:N_CLASSES, 0:1]
    present = cnt > 0.0
    denom = jnp.where(present, cnt, 1.0)
    mu = s / denom
    nbt = nbt_ref[0]
    cen = centers_ref[...]
    out_ref[...] = jnp.where(present, (mu + cen * nbt) / (nbt + 1.0), cen)


_finalize = pl.pallas_call(
    _finalize_body,
    out_shape=jax.ShapeDtypeStruct((N_CLASSES, N_EMB), jnp.float32),
    in_specs=[
        pl.BlockSpec(memory_space=pltpu.SMEM),
        pl.BlockSpec(memory_space=pltpu.ANY),
        pl.BlockSpec(memory_space=pltpu.ANY),
        pl.BlockSpec(memory_space=pltpu.ANY),
    ],
)


def kernel(x, y, centers, num_batches_tracked):
    y2 = y.reshape(BATCH // CHUNK, CHUNK)
    sums, cnts = _scatter_add(x, y2)
    new_centers = _finalize(num_batches_tracked, sums, cnts, centers)
    return (x, new_centers)


# speedup vs baseline: 4.0806x; 4.0806x over previous
import jax
import jax.numpy as jnp
from jax.experimental import pallas as pl
from jax.experimental.pallas import tpu as pltpu

N_CLASSES = 1000
N_EMB = 64
BATCH = 16384
BCHUNK = 2048
NSTEPS = BATCH // BCHUNK


def _body(nbt_ref, x_ref, y_ref, centers_ref, out_ref, acc_ref):
    step = pl.program_id(0)

    @pl.when(step == 0)
    def _():
        acc_ref[...] = jnp.zeros_like(acc_ref)

    x = x_ref[...]
    y = y_ref[...]
    ids = jax.lax.broadcasted_iota(jnp.int32, (BCHUNK, N_CLASSES), 1)
    onehot = jnp.where(y == ids, 1.0, 0.0)
    xa = jnp.concatenate(
        [x, jnp.ones((BCHUNK, 1), jnp.float32),
         jnp.zeros((BCHUNK, 63), jnp.float32)], axis=1)
    acc_ref[...] += jax.lax.dot_general(
        onehot, xa, (((0,), (0,)), ((), ())),
        preferred_element_type=jnp.float32)

    @pl.when(step == NSTEPS - 1)
    def _():
        acc = acc_ref[...]
        s = acc[:, :N_EMB]
        cnt = acc[:, N_EMB:N_EMB + 1]
        present = cnt > 0.0
        denom = jnp.where(present, cnt, 1.0)
        mu = s / denom
        nbt = nbt_ref[0]
        cen = centers_ref[...]
        out_ref[...] = jnp.where(present, (mu + cen * nbt) / (nbt + 1.0), cen)


_seg_update = pl.pallas_call(
    _body,
    grid=(NSTEPS,),
    out_shape=jax.ShapeDtypeStruct((N_CLASSES, N_EMB), jnp.float32),
    in_specs=[
        pl.BlockSpec(memory_space=pltpu.SMEM),
        pl.BlockSpec((BCHUNK, N_EMB), lambda i: (i, 0)),
        pl.BlockSpec((BCHUNK, 1), lambda i: (i, 0)),
        pl.BlockSpec((N_CLASSES, N_EMB), lambda i: (0, 0)),
    ],
    out_specs=pl.BlockSpec((N_CLASSES, N_EMB), lambda i: (0, 0)),
    scratch_shapes=[pltpu.VMEM((N_CLASSES, 128), jnp.float32)],
)


def kernel(x, y, centers, num_batches_tracked):
    new_centers = _seg_update(num_batches_tracked, x, y.reshape(BATCH, 1),
                              centers)
    return (x, new_centers)


# speedup vs baseline: 4.1242x; 1.0107x over previous
import jax
import jax.numpy as jnp
from jax.experimental import pallas as pl
from jax.experimental.pallas import tpu as pltpu

N_CLASSES = 1000
N_EMB = 64
BATCH = 16384
BCHUNK = 2048
NSTEPS = BATCH // BCHUNK


def _body(nbt_ref, x_ref, y_ref, centers_ref, out_ref, acc_ref):
    step = pl.program_id(0)

    @pl.when(step == 0)
    def _():
        acc_ref[...] = jnp.zeros_like(acc_ref)

    x = x_ref[...]
    y = y_ref[...]
    ids = jax.lax.broadcasted_iota(jnp.int32, (BCHUNK, N_CLASSES), 1)
    onehot = (y == ids).astype(jnp.bfloat16)
    xh = x.astype(jnp.bfloat16)
    xl = (x - xh.astype(jnp.float32)).astype(jnp.bfloat16)
    xa = jnp.concatenate(
        [xh, xl, jnp.ones((BCHUNK, 1), jnp.bfloat16),
         jnp.zeros((BCHUNK, 63), jnp.bfloat16)], axis=1)
    acc_ref[...] += jax.lax.dot_general(
        onehot, xa, (((0,), (0,)), ((), ())),
        preferred_element_type=jnp.float32)

    @pl.when(step == NSTEPS - 1)
    def _():
        acc = acc_ref[...]
        s = acc[:, :N_EMB] + acc[:, N_EMB:2 * N_EMB]
        cnt = acc[:, 2 * N_EMB:2 * N_EMB + 1]
        present = cnt > 0.0
        denom = jnp.where(present, cnt, 1.0)
        mu = s / denom
        nbt = nbt_ref[0]
        cen = centers_ref[...]
        out_ref[...] = jnp.where(present, (mu + cen * nbt) / (nbt + 1.0), cen)


_seg_update = pl.pallas_call(
    _body,
    grid=(NSTEPS,),
    out_shape=jax.ShapeDtypeStruct((N_CLASSES, N_EMB), jnp.float32),
    in_specs=[
        pl.BlockSpec(memory_space=pltpu.SMEM),
        pl.BlockSpec((BCHUNK, N_EMB), lambda i: (i, 0)),
        pl.BlockSpec((BCHUNK, 1), lambda i: (i, 0)),
        pl.BlockSpec((N_CLASSES, N_EMB), lambda i: (0, 0)),
    ],
    out_specs=pl.BlockSpec((N_CLASSES, N_EMB), lambda i: (0, 0)),
    scratch_shapes=[pltpu.VMEM((N_CLASSES, 192), jnp.float32)],
)


def kernel(x, y, centers, num_batches_tracked):
    new_centers = _seg_update(num_batches_tracked, x, y.reshape(BATCH, 1),
                              centers)
    return (x, new_centers)


# speedup vs baseline: 5.5191x; 1.3382x over previous
import jax
import jax.numpy as jnp
from jax.experimental import pallas as pl
from jax.experimental.pallas import tpu as pltpu

N_CLASSES = 1000
N_EMB = 64
BATCH = 16384
BCHUNK = 2048
NSTEPS = BATCH // BCHUNK


def _body(nbt_ref, x_ref, y_ref, centers_ref, out_ref, acc_ref):
    step = pl.program_id(0)

    @pl.when(step == 0)
    def _():
        acc_ref[...] = jnp.zeros_like(acc_ref)

    x = x_ref[...]
    y = y_ref[...]
    ids = jax.lax.broadcasted_iota(jnp.int32, (N_CLASSES, BCHUNK), 0)
    onehot_t = (y == ids).astype(jnp.bfloat16)
    xh = x.astype(jnp.bfloat16)
    xl = (x - xh.astype(jnp.float32)).astype(jnp.bfloat16)
    xa = jnp.concatenate(
        [xh, xl, jnp.ones((BCHUNK, 1), jnp.bfloat16),
         jnp.zeros((BCHUNK, 63), jnp.bfloat16)], axis=1)
    acc_ref[...] += jax.lax.dot_general(
        onehot_t, xa, (((1,), (0,)), ((), ())),
        preferred_element_type=jnp.float32)

    @pl.when(step == NSTEPS - 1)
    def _():
        acc = acc_ref[...]
        s = acc[:, :N_EMB] + acc[:, N_EMB:2 * N_EMB]
        cnt = acc[:, 2 * N_EMB:2 * N_EMB + 1]
        present = cnt > 0.0
        denom = jnp.where(present, cnt, 1.0)
        mu = s / denom
        nbt = nbt_ref[0]
        cen = centers_ref[...]
        out_ref[...] = jnp.where(present, (mu + cen * nbt) / (nbt + 1.0), cen)


_seg_update = pl.pallas_call(
    _body,
    grid=(NSTEPS,),
    out_shape=jax.ShapeDtypeStruct((N_CLASSES, N_EMB), jnp.float32),
    in_specs=[
        pl.BlockSpec(memory_space=pltpu.SMEM),
        pl.BlockSpec((BCHUNK, N_EMB), lambda i: (i, 0)),
        pl.BlockSpec((1, BCHUNK), lambda i: (0, i)),
        pl.BlockSpec((N_CLASSES, N_EMB), lambda i: (0, 0)),
    ],
    out_specs=pl.BlockSpec((N_CLASSES, N_EMB), lambda i: (0, 0)),
    scratch_shapes=[pltpu.VMEM((N_CLASSES, 192), jnp.float32)],
)


def kernel(x, y, centers, num_batches_tracked):
    new_centers = _seg_update(num_batches_tracked, x, y.reshape(1, BATCH),
                              centers)
    return (x, new_centers)


# speedup vs baseline: 5.6794x; 1.0290x over previous
import jax
import jax.numpy as jnp
from jax.experimental import pallas as pl
from jax.experimental.pallas import tpu as pltpu

N_CLASSES = 1000
N_EMB = 64
BATCH = 16384
BCHUNK = 4096
NSTEPS = BATCH // BCHUNK


def _body(nbt_ref, x_ref, y_ref, centers_ref, out_ref, acc_ref):
    step = pl.program_id(0)

    @pl.when(step == 0)
    def _():
        acc_ref[...] = jnp.zeros_like(acc_ref)

    x = x_ref[...]
    y = y_ref[...]
    ids = jax.lax.broadcasted_iota(jnp.int32, (N_CLASSES, BCHUNK), 0)
    onehot_t = (y == ids).astype(jnp.bfloat16)
    xh = x.astype(jnp.bfloat16)
    xl = (x - xh.astype(jnp.float32)).astype(jnp.bfloat16)
    xa = jnp.concatenate(
        [xh, xl, jnp.ones((BCHUNK, 1), jnp.bfloat16),
         jnp.zeros((BCHUNK, 63), jnp.bfloat16)], axis=1)
    acc_ref[...] += jax.lax.dot_general(
        onehot_t, xa, (((1,), (0,)), ((), ())),
        preferred_element_type=jnp.float32)

    @pl.when(step == NSTEPS - 1)
    def _():
        acc = acc_ref[...]
        s = acc[:, :N_EMB] + acc[:, N_EMB:2 * N_EMB]
        cnt = acc[:, 2 * N_EMB:2 * N_EMB + 1]
        present = cnt > 0.0
        denom = jnp.where(present, cnt, 1.0)
        mu = s / denom
        nbt = nbt_ref[0]
        cen = centers_ref[...]
        out_ref[...] = jnp.where(present, (mu + cen * nbt) / (nbt + 1.0), cen)


_seg_update = pl.pallas_call(
    _body,
    grid=(NSTEPS,),
    out_shape=jax.ShapeDtypeStruct((N_CLASSES, N_EMB), jnp.float32),
    in_specs=[
        pl.BlockSpec(memory_space=pltpu.SMEM),
        pl.BlockSpec((BCHUNK, N_EMB), lambda i: (i, 0)),
        pl.BlockSpec((1, BCHUNK), lambda i: (0, i)),
        pl.BlockSpec((N_CLASSES, N_EMB), lambda i: (0, 0)),
    ],
    out_specs=pl.BlockSpec((N_CLASSES, N_EMB), lambda i: (0, 0)),
    scratch_shapes=[pltpu.VMEM((N_CLASSES, 192), jnp.float32)],
)


def kernel(x, y, centers, num_batches_tracked):
    new_centers = _seg_update(num_batches_tracked, x, y.reshape(1, BATCH),
                              centers)
    return (x, new_centers)
